# Initial kernel scaffold; baseline (speedup 1.0000x reference)
import jax
import jax.numpy as jnp
from jax.experimental import pallas as pl


def kernel(values, x, selector_0_idx, row_end_offsets):
    raise NotImplementedError("write your pallas kernel here")


# speedup vs baseline: 1684.0784x; 1684.0784x over previous
import functools

import jax
import jax.numpy as jnp
from jax import lax
from jax.experimental import pallas as pl
from jax.experimental.pallas import tpu as pltpu
from jax.experimental.pallas import tpu_sc as plsc

NC = 2
NS = 16
NW = NC * NS
L = 16
BLK = 2048


@functools.lru_cache(maxsize=None)
def _build(nnz, num_cols, num_rows):
    c_sz = (-(-nnz // NW) + BLK - 1) // BLK * BLK
    nb = c_sz // BLK
    nnz_pad = NW * c_sz
    r_per = num_rows // NW
    nve = (r_per + 1 + L - 1) // L + (1 if (r_per + 1) % L else 0)
    nve = -(-(r_per + 1) // L)
    ro_per = nve * L
    ro_pad = (NW - 1) * r_per + ro_per

    mesh = plsc.VectorSubcoreMesh(core_axis_name="c", subcore_axis_name="s")
    cparams = pltpu.CompilerParams(needs_layout_passes=False)

    def scan_body(vals_hbm, idx_hbm, x_hbm, g_hbm, tot_hbm,
                  x_v, vbuf, ibuf, obuf, tv):
        wid = lax.axis_index("s") * NC + lax.axis_index("c")
        cs = wid * c_sz
        pltpu.sync_copy(x_hbm, x_v)

        def block(b, carry):
            off = cs + b * BLK
            pltpu.sync_copy(vals_hbm.at[pl.ds(off, BLK)], vbuf)
            pltpu.sync_copy(idx_hbm.at[pl.ds(off, BLK)], ibuf)

            def vstep(i, carry):
                v = vbuf[pl.ds(i * L, L)]
                ix = ibuf[pl.ds(i * L, L)]
                g = plsc.load_gather(x_v, [ix])
                p = v * g
                s = plsc.cumsum(p)
                obuf[pl.ds(i * L, L)] = s + carry
                return carry + jnp.sum(p)

            carry = lax.fori_loop(0, BLK // L, vstep, carry)
            pltpu.sync_copy(obuf, g_hbm.at[pl.ds(off, BLK)])
            return carry

        carry = lax.fori_loop(0, nb, block, jnp.float32(0.0))
        tv[...] = jnp.full((L,), carry, jnp.float32)
        pltpu.sync_copy(tv, tot_hbm.at[wid])

    scan_k = pl.kernel(
        scan_body,
        out_type=(
            jax.ShapeDtypeStruct((nnz_pad,), jnp.float32),
            jax.ShapeDtypeStruct((NW, L), jnp.float32),
        ),
        mesh=mesh,
        compiler_params=cparams,
        scratch_types=[
            pltpu.VMEM((num_cols,), jnp.float32),
            pltpu.VMEM((BLK,), jnp.float32),
            pltpu.VMEM((BLK,), jnp.int32),
            pltpu.VMEM((BLK,), jnp.float32),
            pltpu.VMEM((L,), jnp.float32),
        ],
    )

    def bound_body(ro_hbm, tot_hbm, g_hbm, y_hbm,
                   ro_v, tot_v, off_v, ev, gv, yv, sem):
        wid = lax.axis_index("s") * NC + lax.axis_index("c")
        base = wid * r_per
        pltpu.sync_copy(ro_hbm.at[pl.ds(base, ro_per)], ro_v)
        pltpu.sync_copy(tot_hbm, tot_v)

        i0 = jnp.arange(L, dtype=jnp.int32)
        z16 = jnp.zeros((L,), jnp.int32)
        t0 = plsc.load_gather(tot_v, [i0, z16])
        t1 = plsc.load_gather(tot_v, [i0 + L, z16])
        off_v[pl.ds(0, L)] = plsc.cumsum(t0) - t0
        off_v[pl.ds(L, L)] = plsc.cumsum(t1) - t1 + jnp.sum(t0)

        def estep(k, _):
            p = ro_v[pl.ds(k * L, L)]
            c = p // c_sz
            q = p - c * c_sz
            pm1 = jnp.where(q > 0, p - 1, p)
            cp = pltpu.async_copy(g_hbm.at[pm1], gv, sem)
            offc = plsc.load_gather(off_v, [c])
            cp.wait()
            ev[pl.ds(k * L, L)] = offc + jnp.where(q > 0, gv[...], 0.0)
            return 0

        lax.fori_loop(0, nve, estep, 0)

        def dstep(k, _):
            a = ev[pl.ds(k * L, L)]
            b = plsc.load_gather(ev, [i0 + (k * L + 1)])
            yv[pl.ds(k * L, L)] = b - a
            return 0

        lax.fori_loop(0, r_per // L, dstep, 0)
        pltpu.sync_copy(yv, y_hbm.at[pl.ds(base, r_per)])

    bound_k = pl.kernel(
        bound_body,
        out_type=jax.ShapeDtypeStruct((num_rows,), jnp.float32),
        mesh=mesh,
        compiler_params=cparams,
        scratch_types=[
            pltpu.VMEM((ro_per,), jnp.int32),
            pltpu.VMEM((NW, L), jnp.float32),
            pltpu.VMEM((NW,), jnp.float32),
            pltpu.VMEM((ro_per,), jnp.float32),
            pltpu.VMEM((L,), jnp.float32),
            pltpu.VMEM((r_per,), jnp.float32),
            pltpu.SemaphoreType.DMA,
        ],
    )

    return scan_k, bound_k, nnz_pad, ro_pad


def kernel(values, x, selector_0_idx, row_end_offsets):
    nnz = values.shape[0]
    num_cols = x.shape[0]
    num_rows = row_end_offsets.shape[0] - 1
    scan_k, bound_k, nnz_pad, ro_pad = _build(nnz, num_cols, num_rows)

    pad = nnz_pad - nnz
    values_p = jnp.concatenate([values, jnp.zeros((pad,), values.dtype)])
    idx_p = jnp.concatenate(
        [selector_0_idx, jnp.zeros((pad,), selector_0_idx.dtype)])
    ro_p = jnp.concatenate([
        row_end_offsets,
        jnp.full((ro_pad - (num_rows + 1),), nnz, row_end_offsets.dtype),
    ])

    g, tot = scan_k(values_p, idx_p, x)
    return bound_k(ro_p, tot, g)


# speedup vs baseline: 3764.9155x; 2.2356x over previous
import functools

import jax
import jax.numpy as jnp
from jax import lax
from jax.experimental import pallas as pl
from jax.experimental.pallas import tpu as pltpu
from jax.experimental.pallas import tpu_sc as plsc

NC = 2
NS = 16
NW = NC * NS
L = 16
BLK = 2048


@functools.lru_cache(maxsize=None)
def _build(nnz, num_cols, num_rows):
    c_sz = (-(-nnz // NW) + BLK - 1) // BLK * BLK
    nb = c_sz // BLK
    nnz_pad = NW * c_sz
    tail_base = nnz // BLK * BLK
    r_per = num_rows // NW
    nve = -(-(r_per + 1) // L)
    ro_per = nve * L
    ro_pad = (NW - 1) * r_per + ro_per
    ng = -(-ro_per // 128)

    mesh = plsc.VectorSubcoreMesh(core_axis_name="c", subcore_axis_name="s")
    cparams = pltpu.CompilerParams(needs_layout_passes=False)

    def scan_body(vals_hbm, idx_hbm, x_hbm, tv_hbm, ti_hbm, g_hbm, tot_hbm,
                  x_v, vbuf, ibuf, obuf, tv, sin, sout):
        wid = lax.axis_index("s") * NC + lax.axis_index("c")
        cs = wid * c_sz
        nb_w = (jnp.minimum(c_sz, nnz - cs) + BLK - 1) // BLK

        def issue_in(b, slot):
            off = cs + b * BLK

            @pl.when(off + BLK <= nnz)
            def _():
                pltpu.async_copy(vals_hbm.at[pl.ds(off, BLK)],
                                 vbuf.at[slot], sin.at[slot])
                pltpu.async_copy(idx_hbm.at[pl.ds(off, BLK)],
                                 ibuf.at[slot], sin.at[slot])

            @pl.when(off + BLK > nnz)
            def _():
                pltpu.async_copy(tv_hbm, vbuf.at[slot], sin.at[slot])
                pltpu.async_copy(ti_hbm, ibuf.at[slot], sin.at[slot])

        issue_in(0, 0)
        pltpu.sync_copy(x_hbm, x_v)

        def block(b, carry):
            slot = b & 1

            @pl.when(b + 1 < nb_w)
            def _():
                issue_in(b + 1, 1 - slot)

            pltpu.make_async_copy(vals_hbm.at[pl.ds(0, BLK)],
                                  vbuf.at[slot], sin.at[slot]).wait()
            pltpu.make_async_copy(idx_hbm.at[pl.ds(0, BLK)],
                                  ibuf.at[slot], sin.at[slot]).wait()

            @pl.when(b >= 2)
            def _():
                pltpu.make_async_copy(obuf.at[slot],
                                      g_hbm.at[pl.ds(0, BLK)],
                                      sout.at[slot]).wait()

            @plsc.parallel_loop(0, BLK // L, unroll=8, carry=carry)
            def vstep(i, c):
                v = vbuf[slot, pl.ds(i * L, L)]
                ix = ibuf[slot, pl.ds(i * L, L)]
                p = v * plsc.load_gather(x_v, [ix])
                s = plsc.cumsum(p)
                obuf[slot, pl.ds(i * L, L)] = s + c
                return c + jnp.sum(p)

            off = cs + b * BLK
            pltpu.async_copy(obuf.at[slot], g_hbm.at[pl.ds(off, BLK)],
                             sout.at[slot])
            return vstep

        carry = lax.fori_loop(0, nb_w, block, jnp.float32(0.0))

        @pl.when(nb_w >= 1)
        def _():
            pltpu.make_async_copy(obuf.at[(nb_w - 1) & 1],
                                  g_hbm.at[pl.ds(0, BLK)],
                                  sout.at[(nb_w - 1) & 1]).wait()

        @pl.when(nb_w >= 2)
        def _():
            pltpu.make_async_copy(obuf.at[(nb_w - 2) & 1],
                                  g_hbm.at[pl.ds(0, BLK)],
                                  sout.at[(nb_w - 2) & 1]).wait()

        tv[...] = jnp.full((L,), carry, jnp.float32)
        pltpu.sync_copy(tv, tot_hbm.at[wid])

    scan_k = pl.kernel(
        scan_body,
        out_type=(
            jax.ShapeDtypeStruct((nnz_pad,), jnp.float32),
            jax.ShapeDtypeStruct((NW, L), jnp.float32),
        ),
        mesh=mesh,
        compiler_params=cparams,
        scratch_types=[
            pltpu.VMEM((num_cols,), jnp.float32),
            pltpu.VMEM((2, BLK), jnp.float32),
            pltpu.VMEM((2, BLK), jnp.int32),
            pltpu.VMEM((2, BLK), jnp.float32),
            pltpu.VMEM((L,), jnp.float32),
            pltpu.SemaphoreType.DMA((2,)),
            pltpu.SemaphoreType.DMA((2,)),
        ],
    )

    def bound_body(ro_hbm, tot_hbm, g_hbm, y_hbm,
                   ro_v, tot_v, off_v, ixbuf, gbuf, ev, yv, sem):
        wid = lax.axis_index("s") * NC + lax.axis_index("c")
        base = wid * r_per
        pltpu.sync_copy(ro_hbm.at[pl.ds(base, ro_per)], ro_v)
        pltpu.sync_copy(tot_hbm, tot_v)

        i0 = jnp.arange(L, dtype=jnp.int32)
        z16 = jnp.zeros((L,), jnp.int32)
        t0 = plsc.load_gather(tot_v, [i0, z16])
        t1 = plsc.load_gather(tot_v, [i0 + L, z16])
        off_v[pl.ds(0, L)] = plsc.cumsum(t0) - t0
        off_v[pl.ds(L, L)] = plsc.cumsum(t1) - t1 + jnp.sum(t0)

        for k in range(ng * 8):
            if k < nve:
                p = ro_v[pl.ds(k * L, L)]
                c = p // c_sz
                q = p - c * c_sz
                pm1 = jnp.where(q > 0, p - 1, p)
            else:
                pm1 = z16
            ixbuf[k // 8, pl.ds((k % 8) * L, L)] = pm1

        cps = [
            pltpu.async_copy(g_hbm.at[ixbuf.at[j]],
                             gbuf.at[pl.ds(j * 128, 128)], sem)
            for j in range(ng)
        ]
        for cp in cps:
            cp.wait()

        for k in range(nve):
            p = ro_v[pl.ds(k * L, L)]
            c = p // c_sz
            q = p - c * c_sz
            offc = plsc.load_gather(off_v, [c])
            ev[pl.ds(k * L, L)] = offc + jnp.where(
                q > 0, gbuf[pl.ds(k * L, L)], 0.0)

        for k in range(r_per // L):
            a = ev[pl.ds(k * L, L)]
            b = plsc.load_gather(ev, [i0 + (k * L + 1)])
            yv[pl.ds(k * L, L)] = b - a

        pltpu.sync_copy(yv, y_hbm.at[pl.ds(base, r_per)])

    bound_k = pl.kernel(
        bound_body,
        out_type=jax.ShapeDtypeStruct((num_rows,), jnp.float32),
        mesh=mesh,
        compiler_params=cparams,
        scratch_types=[
            pltpu.VMEM((ro_per,), jnp.int32),
            pltpu.VMEM((NW, L), jnp.float32),
            pltpu.VMEM((NW,), jnp.float32),
            pltpu.VMEM((ng, 128), jnp.int32),
            pltpu.VMEM((ng * 128,), jnp.float32),
            pltpu.VMEM((ro_per,), jnp.float32),
            pltpu.VMEM((r_per,), jnp.float32),
            pltpu.SemaphoreType.DMA,
        ],
    )

    return scan_k, bound_k, tail_base, ro_pad


def kernel(values, x, selector_0_idx, row_end_offsets):
    nnz = values.shape[0]
    num_cols = x.shape[0]
    num_rows = row_end_offsets.shape[0] - 1
    scan_k, bound_k, tail_base, ro_pad = _build(nnz, num_cols, num_rows)

    t_len = nnz - tail_base
    tail_v = jnp.zeros((BLK,), values.dtype).at[:t_len].set(
        lax.dynamic_slice(values, (tail_base,), (t_len,)))
    tail_i = jnp.zeros((BLK,), selector_0_idx.dtype).at[:t_len].set(
        lax.dynamic_slice(selector_0_idx, (tail_base,), (t_len,)))
    ro_p = jnp.concatenate([
        row_end_offsets,
        jnp.full((ro_pad - (num_rows + 1),), nnz, row_end_offsets.dtype),
    ])

    g, tot = scan_k(values, selector_0_idx, x, tail_v, tail_i)
    return bound_k(ro_p, tot, g)


# speedup vs baseline: 4312.9718x; 1.1456x over previous
import functools

import jax
import jax.numpy as jnp
from jax import lax
from jax.experimental import pallas as pl
from jax.experimental.pallas import tpu as pltpu
from jax.experimental.pallas import tpu_sc as plsc

NC = 2
NS = 16
NW = NC * NS
L = 16
BLK = 2048


@functools.lru_cache(maxsize=None)
def _build(nnz, num_cols, num_rows):
    c_sz = (-(-nnz // NW) + BLK - 1) // BLK * BLK
    nnz_pad = NW * c_sz
    tail_base = nnz // BLK * BLK
    t_len = nnz - tail_base
    ntv = -(-t_len // L) if t_len else 0
    r_per = num_rows // NW
    nve = -(-(r_per + 1) // L)
    ro_per = nve * L

    mesh = plsc.VectorSubcoreMesh(core_axis_name="c", subcore_axis_name="s")
    cparams = pltpu.CompilerParams(needs_layout_passes=False)

    def scan_body(vals_hbm, idx_hbm, x_hbm, g_hbm, tot_hbm,
                  x_v, vbuf, ibuf, obuf, tbv, tbi, tsum, sin, sout):
        iota = jnp.arange(L, dtype=jnp.int32)
        wid = lax.axis_index("s") * NC + lax.axis_index("c")
        cs = wid * c_sz
        nb_w = jnp.minimum(c_sz, nnz - cs) // BLK

        def issue_in(b, slot):
            off = cs + b * BLK
            pltpu.async_copy(vals_hbm.at[pl.ds(off, BLK)],
                             vbuf.at[slot], sin.at[slot])
            pltpu.async_copy(idx_hbm.at[pl.ds(off, BLK)],
                             ibuf.at[slot], sin.at[slot])

        @pl.when(nb_w > 0)
        def _():
            issue_in(0, 0)

        pltpu.sync_copy(x_hbm, x_v)

        def block(b, carry):
            slot = b & 1

            @pl.when(b + 1 < nb_w)
            def _():
                issue_in(b + 1, 1 - slot)

            pltpu.make_async_copy(vals_hbm.at[pl.ds(0, BLK)],
                                  vbuf.at[slot], sin.at[slot]).wait()
            pltpu.make_async_copy(idx_hbm.at[pl.ds(0, BLK)],
                                  ibuf.at[slot], sin.at[slot]).wait()

            @pl.when(b >= 2)
            def _():
                pltpu.make_async_copy(obuf.at[slot],
                                      g_hbm.at[pl.ds(0, BLK)],
                                      sout.at[slot]).wait()

            @plsc.parallel_loop(0, BLK // L, unroll=8, carry=carry)
            def vstep(i, c):
                v = vbuf[slot, pl.ds(i * L, L)]
                ix = ibuf[slot, pl.ds(i * L, L)]
                p = v * plsc.load_gather(x_v, [ix])
                s = plsc.cumsum(p)
                obuf[slot, pl.ds(i * L, L)] = s + c
                return c + s[L - 1]

            off = cs + b * BLK
            pltpu.async_copy(obuf.at[slot], g_hbm.at[pl.ds(off, BLK)],
                             sout.at[slot])
            return vstep

        carry = lax.fori_loop(0, nb_w, block, jnp.float32(0.0))

        @pl.when(nb_w >= 1)
        def _():
            pltpu.make_async_copy(obuf.at[(nb_w - 1) & 1],
                                  g_hbm.at[pl.ds(0, BLK)],
                                  sout.at[(nb_w - 1) & 1]).wait()

        @pl.when(nb_w >= 2)
        def _():
            pltpu.make_async_copy(obuf.at[(nb_w - 2) & 1],
                                  g_hbm.at[pl.ds(0, BLK)],
                                  sout.at[(nb_w - 2) & 1]).wait()

        tsum[...] = jnp.full((L,), carry, jnp.float32)

        if t_len:
            @pl.when(wid == NW - 1)
            def _():
                pltpu.sync_copy(vals_hbm.at[pl.ds(tail_base, t_len)], tbv)
                pltpu.sync_copy(idx_hbm.at[pl.ds(tail_base, t_len)], tbi)
                c = carry
                for k in range(ntv):
                    if (k + 1) * L <= t_len:
                        v = tbv[pl.ds(k * L, L)]
                        ix = tbi[pl.ds(k * L, L)]
                    else:
                        lane_ok = iota < (t_len - k * L)
                        src = jnp.minimum(k * L + iota, t_len - 1)
                        v = jnp.where(lane_ok,
                                      plsc.load_gather(tbv, [src]), 0.0)
                        ix = jnp.where(lane_ok,
                                       plsc.load_gather(tbi, [src]), 0)
                    p = v * plsc.load_gather(x_v, [ix])
                    s = plsc.cumsum(p)
                    obuf[0, pl.ds(k * L, L)] = s + c
                    c = c + s[L - 1]
                pltpu.sync_copy(obuf.at[0], g_hbm.at[pl.ds(tail_base, BLK)])
                tsum[...] = jnp.full((L,), c, jnp.float32)

        pltpu.sync_copy(tsum, tot_hbm.at[wid])

    scan_k = pl.kernel(
        scan_body,
        out_type=(
            jax.ShapeDtypeStruct((nnz_pad,), jnp.float32),
            jax.ShapeDtypeStruct((NW, L), jnp.float32),
        ),
        mesh=mesh,
        compiler_params=cparams,
        scratch_types=[
            pltpu.VMEM((num_cols,), jnp.float32),
            pltpu.VMEM((2, BLK), jnp.float32),
            pltpu.VMEM((2, BLK), jnp.int32),
            pltpu.VMEM((2, BLK), jnp.float32),
            pltpu.VMEM((max(t_len, L),), jnp.float32),
            pltpu.VMEM((max(t_len, L),), jnp.int32),
            pltpu.VMEM((L,), jnp.float32),
            pltpu.SemaphoreType.DMA((2,)),
            pltpu.SemaphoreType.DMA((2,)),
        ],
    )

    def bound_body(ro_hbm, tot_hbm, g_hbm, y_hbm, *refs):
        ro_main, ro_tail, tot_v, off_v = refs[0], refs[1], refs[2], refs[3]
        gbufs = refs[4:4 + nve]
        ev, yv, sem, semr = refs[4 + nve:]
        iota = jnp.arange(L, dtype=jnp.int32)
        wid = lax.axis_index("s") * NC + lax.axis_index("c")
        base = wid * r_per
        pltpu.async_copy(ro_hbm.at[pl.ds(base, r_per)], ro_main, semr)
        cl = jnp.minimum(base + r_per + iota, num_rows)
        pltpu.async_copy(ro_hbm.at[cl], ro_tail, semr)
        pltpu.sync_copy(tot_hbm, tot_v)

        z16 = jnp.zeros((L,), jnp.int32)
        t0 = plsc.load_gather(tot_v, [iota, z16])
        t1 = plsc.load_gather(tot_v, [iota + L, z16])
        off_v[pl.ds(0, L)] = plsc.cumsum(t0) - t0
        off_v[pl.ds(L, L)] = plsc.cumsum(t1) - t1 + jnp.sum(t0)

        pltpu.make_async_copy(ro_hbm.at[pl.ds(0, r_per)],
                              ro_main, semr).wait()
        pltpu.make_async_copy(ro_hbm.at[iota], ro_tail, semr).wait()

        def bpos(k):
            if (k + 1) * L <= r_per:
                return ro_main[pl.ds(k * L, L)]
            return ro_tail[...]

        cps = []
        for k in range(nve):
            p = bpos(k)
            c = p // c_sz
            q = p - c * c_sz
            pm1 = jnp.where(q > 0, p - 1, p)
            cps.append(pltpu.async_copy(g_hbm.at[pm1], gbufs[k], sem))
        for cp in cps:
            cp.wait()

        for k in range(nve):
            p = bpos(k)
            c = p // c_sz
            q = p - c * c_sz
            offc = plsc.load_gather(off_v, [c])
            ev[pl.ds(k * L, L)] = offc + jnp.where(q > 0, gbufs[k][...], 0.0)

        for k in range(r_per // L):
            a = ev[pl.ds(k * L, L)]
            b = plsc.load_gather(ev, [iota + (k * L + 1)])
            yv[pl.ds(k * L, L)] = b - a

        pltpu.sync_copy(yv, y_hbm.at[pl.ds(base, r_per)])

    bound_k = pl.kernel(
        bound_body,
        out_type=jax.ShapeDtypeStruct((num_rows,), jnp.float32),
        mesh=mesh,
        compiler_params=cparams,
        scratch_types=[
            pltpu.VMEM((r_per,), jnp.int32),
            pltpu.VMEM((L,), jnp.int32),
            pltpu.VMEM((NW, L), jnp.float32),
            pltpu.VMEM((NW,), jnp.float32),
        ] + [pltpu.VMEM((L,), jnp.float32) for _ in range(nve)] + [
            pltpu.VMEM((ro_per,), jnp.float32),
            pltpu.VMEM((r_per,), jnp.float32),
            pltpu.SemaphoreType.DMA,
            pltpu.SemaphoreType.DMA,
        ],
    )

    return scan_k, bound_k


def kernel(values, x, selector_0_idx, row_end_offsets):
    scan_k, bound_k = _build(values.shape[0], x.shape[0],
                             row_end_offsets.shape[0] - 1)
    g, tot = scan_k(values, selector_0_idx, x)
    return bound_k(row_end_offsets, tot, g)


# speedup vs baseline: 4450.4257x; 1.0319x over previous
import functools

import jax
import jax.numpy as jnp
from jax import lax
from jax.experimental import pallas as pl
from jax.experimental.pallas import tpu as pltpu
from jax.experimental.pallas import tpu_sc as plsc

NC = 2
NS = 16
NW = NC * NS
L = 16
BLK = 2048


@functools.lru_cache(maxsize=None)
def _build(nnz, num_cols, num_rows):
    c_sz = (-(-nnz // NW) + BLK - 1) // BLK * BLK
    nnz_pad = NW * c_sz
    tail_base = nnz // BLK * BLK
    t_len = nnz - tail_base
    ntv = -(-t_len // L) if t_len else 0
    r_per = num_rows // NW
    nve = -(-(r_per + 1) // L)
    ro_per = nve * L

    mesh = plsc.VectorSubcoreMesh(core_axis_name="c", subcore_axis_name="s")
    cparams = pltpu.CompilerParams(needs_layout_passes=False)

    def scan_body(vals_hbm, idx_hbm, x_hbm, g_hbm, tot_hbm,
                  x_v, vbuf, ibuf, obuf, tbv, tbi, tsum, sin, sout):
        iota = jnp.arange(L, dtype=jnp.int32)
        wid = lax.axis_index("s") * NC + lax.axis_index("c")
        cs = wid * c_sz
        nb_w = jnp.minimum(c_sz, nnz - cs) // BLK

        def issue_in(b, slot):
            off = cs + b * BLK
            pltpu.async_copy(vals_hbm.at[pl.ds(off, BLK)],
                             vbuf.at[slot], sin.at[slot])
            pltpu.async_copy(idx_hbm.at[pl.ds(off, BLK)],
                             ibuf.at[slot], sin.at[slot])

        @pl.when(nb_w > 0)
        def _():
            issue_in(0, 0)

        pltpu.sync_copy(x_hbm, x_v)

        def block(b, carry):
            slot = b & 1

            @pl.when(b + 1 < nb_w)
            def _():
                issue_in(b + 1, 1 - slot)

            pltpu.make_async_copy(vals_hbm.at[pl.ds(0, BLK)],
                                  vbuf.at[slot], sin.at[slot]).wait()
            pltpu.make_async_copy(idx_hbm.at[pl.ds(0, BLK)],
                                  ibuf.at[slot], sin.at[slot]).wait()

            @pl.when(b >= 2)
            def _():
                pltpu.make_async_copy(obuf.at[slot],
                                      g_hbm.at[pl.ds(0, BLK)],
                                      sout.at[slot]).wait()

            @plsc.parallel_loop(0, BLK // L, unroll=16, carry=carry)
            def vstep(i, c):
                v = vbuf[slot, pl.ds(i * L, L)]
                ix = ibuf[slot, pl.ds(i * L, L)]
                p = v * plsc.load_gather(x_v, [ix])
                s = plsc.cumsum(p)
                obuf[slot, pl.ds(i * L, L)] = s + c
                return c + s[L - 1]

            off = cs + b * BLK
            pltpu.async_copy(obuf.at[slot], g_hbm.at[pl.ds(off, BLK)],
                             sout.at[slot])
            return vstep

        carry = lax.fori_loop(0, nb_w, block, jnp.float32(0.0))

        @pl.when(nb_w >= 1)
        def _():
            pltpu.make_async_copy(obuf.at[(nb_w - 1) & 1],
                                  g_hbm.at[pl.ds(0, BLK)],
                                  sout.at[(nb_w - 1) & 1]).wait()

        @pl.when(nb_w >= 2)
        def _():
            pltpu.make_async_copy(obuf.at[(nb_w - 2) & 1],
                                  g_hbm.at[pl.ds(0, BLK)],
                                  sout.at[(nb_w - 2) & 1]).wait()

        tsum[...] = jnp.full((L,), carry, jnp.float32)

        if t_len:
            @pl.when(wid == NW - 1)
            def _():
                pltpu.sync_copy(vals_hbm.at[pl.ds(tail_base, t_len)], tbv)
                pltpu.sync_copy(idx_hbm.at[pl.ds(tail_base, t_len)], tbi)
                c = carry
                for k in range(ntv):
                    if (k + 1) * L <= t_len:
                        v = tbv[pl.ds(k * L, L)]
                        ix = tbi[pl.ds(k * L, L)]
                    else:
                        lane_ok = iota < (t_len - k * L)
                        src = jnp.minimum(k * L + iota, t_len - 1)
                        v = jnp.where(lane_ok,
                                      plsc.load_gather(tbv, [src]), 0.0)
                        ix = jnp.where(lane_ok,
                                       plsc.load_gather(tbi, [src]), 0)
                    p = v * plsc.load_gather(x_v, [ix])
                    s = plsc.cumsum(p)
                    obuf[0, pl.ds(k * L, L)] = s + c
                    c = c + s[L - 1]
                pltpu.sync_copy(obuf.at[0], g_hbm.at[pl.ds(tail_base, BLK)])
                tsum[...] = jnp.full((L,), c, jnp.float32)

        pltpu.sync_copy(tsum, tot_hbm.at[wid])

    scan_k = pl.kernel(
        scan_body,
        out_type=(
            jax.ShapeDtypeStruct((nnz_pad,), jnp.float32),
            jax.ShapeDtypeStruct((NW, L), jnp.float32),
        ),
        mesh=mesh,
        compiler_params=cparams,
        scratch_types=[
            pltpu.VMEM((num_cols,), jnp.float32),
            pltpu.VMEM((2, BLK), jnp.float32),
            pltpu.VMEM((2, BLK), jnp.int32),
            pltpu.VMEM((2, BLK), jnp.float32),
            pltpu.VMEM((max(t_len, L),), jnp.float32),
            pltpu.VMEM((max(t_len, L),), jnp.int32),
            pltpu.VMEM((L,), jnp.float32),
            pltpu.SemaphoreType.DMA((2,)),
            pltpu.SemaphoreType.DMA((2,)),
        ],
    )

    def bound_body(ro_hbm, tot_hbm, g_hbm, y_hbm, *refs):
        ro_main, ro_tail, tot_v, off_v = refs[0], refs[1], refs[2], refs[3]
        gbufs = refs[4:4 + nve]
        ev, yv, sem, semr = refs[4 + nve:]
        iota = jnp.arange(L, dtype=jnp.int32)
        wid = lax.axis_index("s") * NC + lax.axis_index("c")
        base = wid * r_per
        pltpu.async_copy(ro_hbm.at[pl.ds(base, r_per)], ro_main, semr)
        cl = jnp.minimum(base + r_per + iota, num_rows)
        pltpu.async_copy(ro_hbm.at[cl], ro_tail, semr)
        pltpu.sync_copy(tot_hbm, tot_v)

        z16 = jnp.zeros((L,), jnp.int32)
        t0 = plsc.load_gather(tot_v, [iota, z16])
        t1 = plsc.load_gather(tot_v, [iota + L, z16])
        off_v[pl.ds(0, L)] = plsc.cumsum(t0) - t0
        off_v[pl.ds(L, L)] = plsc.cumsum(t1) - t1 + jnp.sum(t0)

        pltpu.make_async_copy(ro_hbm.at[pl.ds(0, r_per)],
                              ro_main, semr).wait()
        pltpu.make_async_copy(ro_hbm.at[iota], ro_tail, semr).wait()

        def bpos(k):
            if (k + 1) * L <= r_per:
                return ro_main[pl.ds(k * L, L)]
            return ro_tail[...]

        cps = []
        for k in range(nve):
            p = bpos(k)
            c = p // c_sz
            q = p - c * c_sz
            pm1 = jnp.where(q > 0, p - 1, p)
            cps.append(pltpu.async_copy(g_hbm.at[pm1], gbufs[k], sem))
        for cp in cps:
            cp.wait()

        for k in range(nve):
            p = bpos(k)
            c = p // c_sz
            q = p - c * c_sz
            offc = plsc.load_gather(off_v, [c])
            ev[pl.ds(k * L, L)] = offc + jnp.where(q > 0, gbufs[k][...], 0.0)

        for k in range(r_per // L):
            a = ev[pl.ds(k * L, L)]
            b = plsc.load_gather(ev, [iota + (k * L + 1)])
            yv[pl.ds(k * L, L)] = b - a

        pltpu.sync_copy(yv, y_hbm.at[pl.ds(base, r_per)])

    bound_k = pl.kernel(
        bound_body,
        out_type=jax.ShapeDtypeStruct((num_rows,), jnp.float32),
        mesh=mesh,
        compiler_params=cparams,
        scratch_types=[
            pltpu.VMEM((r_per,), jnp.int32),
            pltpu.VMEM((L,), jnp.int32),
            pltpu.VMEM((NW, L), jnp.float32),
            pltpu.VMEM((NW,), jnp.float32),
        ] + [pltpu.VMEM((L,), jnp.float32) for _ in range(nve)] + [
            pltpu.VMEM((ro_per,), jnp.float32),
            pltpu.VMEM((r_per,), jnp.float32),
            pltpu.SemaphoreType.DMA,
            pltpu.SemaphoreType.DMA,
        ],
    )

    return scan_k, bound_k


def kernel(values, x, selector_0_idx, row_end_offsets):
    scan_k, bound_k = _build(values.shape[0], x.shape[0],
                             row_end_offsets.shape[0] - 1)
    g, tot = scan_k(values, selector_0_idx, x)
    return bound_k(row_end_offsets, tot, g)


# speedup vs baseline: 4532.6910x; 1.0185x over previous
import functools

import jax
import jax.numpy as jnp
from jax import lax
from jax.experimental import pallas as pl
from jax.experimental.pallas import tpu as pltpu
from jax.experimental.pallas import tpu_sc as plsc

NC = 2
NS = 16
NW = NC * NS
L = 16
BLK = 4096


@functools.lru_cache(maxsize=None)
def _build(nnz, num_cols, num_rows):
    c_sz = (-(-nnz // NW) + BLK - 1) // BLK * BLK
    nnz_pad = NW * c_sz
    tail_base = nnz // BLK * BLK
    t_len = nnz - tail_base
    ntv = -(-t_len // L) if t_len else 0
    r_per = num_rows // NW
    nve = -(-(r_per + 1) // L)
    ro_per = nve * L

    mesh = plsc.VectorSubcoreMesh(core_axis_name="c", subcore_axis_name="s")
    cparams = pltpu.CompilerParams(needs_layout_passes=False)

    def scan_body(vals_hbm, idx_hbm, x_hbm, g_hbm, tot_hbm,
                  x_v, vbuf, ibuf, obuf, tbv, tbi, tsum, sin, sout):
        iota = jnp.arange(L, dtype=jnp.int32)
        wid = lax.axis_index("s") * NC + lax.axis_index("c")
        cs = wid * c_sz
        nb_w = jnp.minimum(c_sz, nnz - cs) // BLK

        def issue_in(b, slot):
            off = cs + b * BLK
            pltpu.async_copy(vals_hbm.at[pl.ds(off, BLK)],
                             vbuf.at[slot], sin.at[slot])
            pltpu.async_copy(idx_hbm.at[pl.ds(off, BLK)],
                             ibuf.at[slot], sin.at[slot])

        @pl.when(nb_w > 0)
        def _():
            issue_in(0, 0)

        pltpu.sync_copy(x_hbm, x_v)

        def block(b, carry):
            slot = b & 1

            @pl.when(b + 1 < nb_w)
            def _():
                issue_in(b + 1, 1 - slot)

            pltpu.make_async_copy(vals_hbm.at[pl.ds(0, BLK)],
                                  vbuf.at[slot], sin.at[slot]).wait()
            pltpu.make_async_copy(idx_hbm.at[pl.ds(0, BLK)],
                                  ibuf.at[slot], sin.at[slot]).wait()

            @pl.when(b >= 2)
            def _():
                pltpu.make_async_copy(obuf.at[slot],
                                      g_hbm.at[pl.ds(0, BLK)],
                                      sout.at[slot]).wait()

            @plsc.parallel_loop(0, BLK // L, unroll=16, carry=carry)
            def vstep(i, c):
                v = vbuf[slot, pl.ds(i * L, L)]
                ix = ibuf[slot, pl.ds(i * L, L)]
                p = v * plsc.load_gather(x_v, [ix])
                s = plsc.cumsum(p)
                obuf[slot, pl.ds(i * L, L)] = s + c
                return c + s[L - 1]

            off = cs + b * BLK
            pltpu.async_copy(obuf.at[slot], g_hbm.at[pl.ds(off, BLK)],
                             sout.at[slot])
            return vstep

        carry = lax.fori_loop(0, nb_w, block, jnp.float32(0.0))

        @pl.when(nb_w >= 1)
        def _():
            pltpu.make_async_copy(obuf.at[(nb_w - 1) & 1],
                                  g_hbm.at[pl.ds(0, BLK)],
                                  sout.at[(nb_w - 1) & 1]).wait()

        @pl.when(nb_w >= 2)
        def _():
            pltpu.make_async_copy(obuf.at[(nb_w - 2) & 1],
                                  g_hbm.at[pl.ds(0, BLK)],
                                  sout.at[(nb_w - 2) & 1]).wait()

        tsum[...] = jnp.full((L,), carry, jnp.float32)

        if t_len:
            @pl.when(wid == NW - 1)
            def _():
                pltpu.sync_copy(vals_hbm.at[pl.ds(tail_base, t_len)], tbv)
                pltpu.sync_copy(idx_hbm.at[pl.ds(tail_base, t_len)], tbi)
                c = carry
                for k in range(ntv):
                    if (k + 1) * L <= t_len:
                        v = tbv[pl.ds(k * L, L)]
                        ix = tbi[pl.ds(k * L, L)]
                    else:
                        lane_ok = iota < (t_len - k * L)
                        src = jnp.minimum(k * L + iota, t_len - 1)
                        v = jnp.where(lane_ok,
                                      plsc.load_gather(tbv, [src]), 0.0)
                        ix = jnp.where(lane_ok,
                                       plsc.load_gather(tbi, [src]), 0)
                    p = v * plsc.load_gather(x_v, [ix])
                    s = plsc.cumsum(p)
                    obuf[0, pl.ds(k * L, L)] = s + c
                    c = c + s[L - 1]
                pltpu.sync_copy(obuf.at[0], g_hbm.at[pl.ds(tail_base, BLK)])
                tsum[...] = jnp.full((L,), c, jnp.float32)

        pltpu.sync_copy(tsum, tot_hbm.at[wid])

    scan_k = pl.kernel(
        scan_body,
        out_type=(
            jax.ShapeDtypeStruct((nnz_pad,), jnp.float32),
            jax.ShapeDtypeStruct((NW, L), jnp.float32),
        ),
        mesh=mesh,
        compiler_params=cparams,
        scratch_types=[
            pltpu.VMEM((num_cols,), jnp.float32),
            pltpu.VMEM((2, BLK), jnp.float32),
            pltpu.VMEM((2, BLK), jnp.int32),
            pltpu.VMEM((2, BLK), jnp.float32),
            pltpu.VMEM((max(t_len, L),), jnp.float32),
            pltpu.VMEM((max(t_len, L),), jnp.int32),
            pltpu.VMEM((L,), jnp.float32),
            pltpu.SemaphoreType.DMA((2,)),
            pltpu.SemaphoreType.DMA((2,)),
        ],
    )

    def bound_body(ro_hbm, tot_hbm, g_hbm, y_hbm, *refs):
        ro_main, ro_tail, tot_v, off_v = refs[0], refs[1], refs[2], refs[3]
        gbufs = refs[4:4 + nve]
        ev, yv, sem, semr = refs[4 + nve:]
        iota = jnp.arange(L, dtype=jnp.int32)
        wid = lax.axis_index("s") * NC + lax.axis_index("c")
        base = wid * r_per
        pltpu.async_copy(ro_hbm.at[pl.ds(base, r_per)], ro_main, semr)
        cl = jnp.minimum(base + r_per + iota, num_rows)
        pltpu.async_copy(ro_hbm.at[cl], ro_tail, semr)
        pltpu.sync_copy(tot_hbm, tot_v)

        z16 = jnp.zeros((L,), jnp.int32)
        t0 = plsc.load_gather(tot_v, [iota, z16])
        t1 = plsc.load_gather(tot_v, [iota + L, z16])
        off_v[pl.ds(0, L)] = plsc.cumsum(t0) - t0
        off_v[pl.ds(L, L)] = plsc.cumsum(t1) - t1 + jnp.sum(t0)

        pltpu.make_async_copy(ro_hbm.at[pl.ds(0, r_per)],
                              ro_main, semr).wait()
        pltpu.make_async_copy(ro_hbm.at[iota], ro_tail, semr).wait()

        def bpos(k):
            if (k + 1) * L <= r_per:
                return ro_main[pl.ds(k * L, L)]
            return ro_tail[...]

        cps = []
        for k in range(nve):
            p = bpos(k)
            c = p // c_sz
            q = p - c * c_sz
            pm1 = jnp.where(q > 0, p - 1, p)
            cps.append(pltpu.async_copy(g_hbm.at[pm1], gbufs[k], sem))
        for cp in cps:
            cp.wait()

        for k in range(nve):
            p = bpos(k)
            c = p // c_sz
            q = p - c * c_sz
            offc = plsc.load_gather(off_v, [c])
            ev[pl.ds(k * L, L)] = offc + jnp.where(q > 0, gbufs[k][...], 0.0)

        for k in range(r_per // L):
            a = ev[pl.ds(k * L, L)]
            b = plsc.load_gather(ev, [iota + (k * L + 1)])
            yv[pl.ds(k * L, L)] = b - a

        pltpu.sync_copy(yv, y_hbm.at[pl.ds(base, r_per)])

    bound_k = pl.kernel(
        bound_body,
        out_type=jax.ShapeDtypeStruct((num_rows,), jnp.float32),
        mesh=mesh,
        compiler_params=cparams,
        scratch_types=[
            pltpu.VMEM((r_per,), jnp.int32),
            pltpu.VMEM((L,), jnp.int32),
            pltpu.VMEM((NW, L), jnp.float32),
            pltpu.VMEM((NW,), jnp.float32),
        ] + [pltpu.VMEM((L,), jnp.float32) for _ in range(nve)] + [
            pltpu.VMEM((ro_per,), jnp.float32),
            pltpu.VMEM((r_per,), jnp.float32),
            pltpu.SemaphoreType.DMA,
            pltpu.SemaphoreType.DMA,
        ],
    )

    return scan_k, bound_k


def kernel(values, x, selector_0_idx, row_end_offsets):
    scan_k, bound_k = _build(values.shape[0], x.shape[0],
                             row_end_offsets.shape[0] - 1)
    g, tot = scan_k(values, selector_0_idx, x)
    return bound_k(row_end_offsets, tot, g)
